# baseline (speedup 1.0000x reference)
import functools

import jax
import jax.numpy as jnp
from jax.experimental import pallas as pl
from jax.experimental.pallas import tpu as pltpu

_K = 8
_THRESHOLD = 0.1
_NEG_INF = float("-inf")


def _body(nse_ref, ne_ref, ur_ref, hm_ref, hmk_ref, pt_ref, pm_ref, kid_ref):
    x = nse_ref[:, :, 1:, :]
    ne = ne_ref[:, :, 1:, :]
    bb, h, sm1, d = x.shape

    q = ur_ref[:, 0, :]
    qn = q / jnp.maximum(jnp.abs(q), 1e-12)

    norm = jnp.sqrt(jnp.sum(x * x, axis=-1, keepdims=True))
    xn = x / jnp.maximum(norm, 1e-12)
    scores = jnp.sum(xn * qn[:, None, None, :], axis=-1)

    hmk = hmk_ref[:, :, 1:] != 0
    scores = jnp.where(hmk, scores, _NEG_INF)
    hm = hm_ref[:, :, 1:]

    s_iota = jax.lax.broadcasted_iota(jnp.int32, (bb, h, sm1), 2)
    for k in range(_K):
        m = jnp.max(scores, axis=-1)
        hit = scores == m[:, :, None]
        idx = jnp.min(jnp.where(hit, s_iota, sm1), axis=-1)
        onehot = s_iota == idx[:, :, None]
        gm = jnp.max(jnp.where(onehot, hm, 0), axis=-1)
        keep = m >= _THRESHOLD
        scale = jnp.where(keep, m, 0.0)
        w = jnp.where(onehot, scale[:, :, None], 0.0)
        pt_ref[:, :, k, :] = jnp.sum(w[..., None] * ne, axis=2)
        pm_ref[:, :, k] = ((gm != 0) & keep).astype(jnp.int32)
        kid_ref[:, :, k] = idx
        scores = jnp.where(onehot, _NEG_INF, scores)


@functools.partial(jax.jit, static_argnames=("bb",))
def _run(nse, ne, ur, hm, hmk, bb=4):
    b, h, s, d = nse.shape
    grid = (b // bb,)
    out_shapes = (
        jax.ShapeDtypeStruct((b, h, _K, d), jnp.float32),
        jax.ShapeDtypeStruct((b, h, _K), jnp.int32),
        jax.ShapeDtypeStruct((b, h, _K), jnp.int32),
    )
    in_specs = [
        pl.BlockSpec((bb, h, s, d), lambda i: (i, 0, 0, 0)),
        pl.BlockSpec((bb, h, s, d), lambda i: (i, 0, 0, 0)),
        pl.BlockSpec((bb, 1, d), lambda i: (i, 0, 0)),
        pl.BlockSpec((bb, h, s), lambda i: (i, 0, 0)),
        pl.BlockSpec((bb, h, s), lambda i: (i, 0, 0)),
    ]
    out_specs = (
        pl.BlockSpec((bb, h, _K, d), lambda i: (i, 0, 0, 0)),
        pl.BlockSpec((bb, h, _K), lambda i: (i, 0, 0)),
        pl.BlockSpec((bb, h, _K), lambda i: (i, 0, 0)),
    )
    return pl.pallas_call(
        _body,
        grid=grid,
        in_specs=in_specs,
        out_specs=out_specs,
        out_shape=out_shapes,
    )(nse, ne, ur, hm, hmk)


def kernel(news_selection_embedding, news_embedding, user_repr, news_repr,
           his_attn_mask, his_attn_mask_k):
    del news_repr
    hm = his_attn_mask.astype(jnp.int32)
    hmk = his_attn_mask_k.astype(jnp.int32)
    ps_terms, pm, kid = _run(news_selection_embedding, news_embedding,
                             user_repr, hm, hmk)
    return ps_terms, pm != 0, kid


# speedup vs baseline: 1.0046x; 1.0046x over previous
import functools

import jax
import jax.numpy as jnp
from jax import lax
from jax.experimental import pallas as pl
from jax.experimental.pallas import tpu as pltpu
from jax.experimental.pallas import tpu_sc as plsc

_K = 8
_THRESHOLD = 0.1
_NEG_INF = float("-inf")

_NC = 2
_NS = 16
_NW = _NC * _NS
_CH = 80


def _score_body(x_ref, ur_ref, hm_ref, hmk_ref,
                kid_ref, pm_ref, scl_ref, fidx_ref):
    _, hs, d = x_ref.shape
    _, h, s = hm_ref.shape
    b = pl.program_id(0)

    x = x_ref[0]
    q = ur_ref[0]
    qn = q / jnp.maximum(jnp.abs(q), 1e-12)

    dims = (((1,), (1,)), ((), ()))
    dot = lax.dot_general(x, qn, dims,
                          preferred_element_type=jnp.float32)
    sumsq = lax.dot_general(x * x, jnp.ones((1, d), jnp.float32), dims,
                            preferred_element_type=jnp.float32)
    scores = dot / jnp.maximum(jnp.sqrt(sumsq), 1e-12)
    scores = jnp.reshape(scores, (h, s))

    s_iota = jax.lax.broadcasted_iota(jnp.int32, (h, s), 1)
    h_iota = jax.lax.broadcasted_iota(jnp.int32, (h,), 0)
    valid = (hmk_ref[0] != 0) & (s_iota >= 1)
    scores = jnp.where(valid, scores, _NEG_INF)
    hm = hm_ref[0]

    for k in range(_K):
        m = jnp.max(scores, axis=-1)
        hit = (scores == m[:, None]) & (s_iota >= 1)
        idx = jnp.min(jnp.where(hit, s_iota, s), axis=-1)
        onehot = s_iota == idx[:, None]
        gm = jnp.max(jnp.where(onehot, hm, 0), axis=-1)
        keep = m >= _THRESHOLD
        kid_ref[0, :, k] = idx - 1
        pm_ref[0, :, k] = ((gm != 0) & keep).astype(jnp.int32)
        scale = jnp.where(keep, m, 0.0)
        scl_ref[0, :, k, :] = jnp.broadcast_to(scale[:, None], (h, 16))
        fidx_ref[0, :, k] = (b * h + h_iota) * s + idx
        scores = jnp.where(onehot, _NEG_INF, scores)


@jax.jit
def _score_topk(nse2, ur, hm, hmk):
    b, hs, d = nse2.shape
    _, h, s = hm.shape
    out_shapes = (
        jax.ShapeDtypeStruct((b, h, _K), jnp.int32),
        jax.ShapeDtypeStruct((b, h, _K), jnp.int32),
        jax.ShapeDtypeStruct((b, h, _K, 16), jnp.float32),
        jax.ShapeDtypeStruct((b, h, _K), jnp.int32),
    )
    spec3 = pl.BlockSpec((1, h, _K), lambda i: (i, 0, 0))
    spec4 = pl.BlockSpec((1, h, _K, 16), lambda i: (i, 0, 0, 0))
    return pl.pallas_call(
        _score_body,
        grid=(b,),
        in_specs=[
            pl.BlockSpec((1, hs, d), lambda i: (i, 0, 0)),
            pl.BlockSpec((1, 1, d), lambda i: (i, 0, 0)),
            pl.BlockSpec((1, h, s), lambda i: (i, 0, 0)),
            pl.BlockSpec((1, h, s), lambda i: (i, 0, 0)),
        ],
        out_specs=(spec3, spec3, spec4, spec3),
        out_shape=out_shapes,
    )(nse2, ur, hm, hmk)


def _sc_gather_body(table, fidx, scl, out, idx_v, scl_v, rows_v, sem):
    rpw = scl_v.shape[0]
    ng = rpw // _CH
    wid = lax.axis_index("s") * _NC + lax.axis_index("c")
    base = wid * rpw
    pltpu.sync_copy(fidx.at[wid], idx_v)
    pltpu.sync_copy(scl.at[wid], scl_v)
    copies = [
        pltpu.async_copy(table.at[idx_v.at[g]],
                         rows_v.at[pl.ds(g * _CH, _CH)], sem)
        for g in range(ng)
    ]
    for cp in copies:
        cp.wait()

    @pl.loop(0, rpw, unroll=4)
    def _mul(r):
        sv = scl_v[r, :]
        for c in range(4):
            sl = pl.ds(c * 16, 16)
            rows_v[r, sl] = rows_v[r, sl] * sv

    pltpu.sync_copy(rows_v, out.at[pl.ds(base, rpw)])


@jax.jit
def _sc_gather_scale(table, fidx2, scl):
    rpw = scl.shape[1]
    n = _NW * rpw
    d = table.shape[-1]
    mesh = plsc.VectorSubcoreMesh(core_axis_name="c", subcore_axis_name="s")
    f = pl.kernel(
        _sc_gather_body,
        out_type=jax.ShapeDtypeStruct((n, d), jnp.float32),
        mesh=mesh,
        scratch_types=[
            pltpu.VMEM((rpw // _CH, _CH), jnp.int32),
            pltpu.VMEM((rpw, 16), jnp.float32),
            pltpu.VMEM((rpw, d), jnp.float32),
            pltpu.SemaphoreType.DMA,
        ],
        compiler_params=pltpu.CompilerParams(use_tc_tiling_on_sc=False),
    )
    return f(table, fidx2, scl)


def kernel(news_selection_embedding, news_embedding, user_repr, news_repr,
           his_attn_mask, his_attn_mask_k):
    del news_repr
    b, h, s, d = news_selection_embedding.shape
    nse2 = jnp.reshape(news_selection_embedding, (b, h * s, d))
    hm = his_attn_mask.astype(jnp.int32)
    hmk = his_attn_mask_k.astype(jnp.int32)

    kid, pm, scl, fidx = _score_topk(nse2, user_repr, hm, hmk)

    n = b * h * _K
    table = jnp.reshape(news_embedding, (b * h * s, d))
    fidx2 = jnp.reshape(fidx, (_NW, n // _NW // _CH, _CH))
    rows = _sc_gather_scale(table, fidx2, jnp.reshape(scl, (_NW, n // _NW, 16)))

    ps_terms = jnp.reshape(rows, (b, h, _K, d))
    return ps_terms, pm != 0, kid
